# baseline (speedup 1.0000x reference)
import functools

import jax
import jax.numpy as jnp
from jax import lax
from jax.experimental import pallas as pl
from jax.experimental.pallas import tpu as pltpu
from jax.experimental.pallas import tpu_sc as plsc

H = W = 48
P = H * W
B = 16
R = B * P
C = 64
PAD = 64
_EB = 2


def _enc_body(x_ref, w1_ref, b1_ref, w2_ref, b2_ref, w3_ref, b3_ref,
              out_ref, p1_ref, p0_ref):
    col = lax.broadcasted_iota(jnp.int32, (P, 1), 0) % W
    mask_l = (col >= 1).astype(jnp.float32)
    mask_r = (col <= W - 2).astype(jnp.float32)

    def conv1(p_ref, w_ref, bias):
        acc = jnp.zeros((P, C), jnp.float32)
        for dx in (-1, 0, 1):
            s = jnp.zeros((P, C), jnp.float32)
            for dy in (-1, 0, 1):
                t = (dy + 1) * 3 + (dx + 1)
                sl = p_ref[pl.ds(PAD + dy * W + dx, P), :]
                s = s + jnp.dot(sl, w_ref[t], preferred_element_type=jnp.float32)
            if dx == -1:
                s = s * mask_l
            elif dx == 1:
                s = s * mask_r
            acc = acc + s
        return acc + bias

    def conv_big(w_ref, bias):
        acc = jnp.zeros((P, C), jnp.float32)
        for dx in (-1, 0, 1):
            s = jnp.zeros((P, C), jnp.float32)
            for dy in (-1, 0, 1):
                t = (dy + 1) * 3 + (dx + 1)
                sl = p0_ref[pl.ds(PAD + dy * W + dx, P), :].astype(jnp.bfloat16)
                wt = w_ref[t].astype(jnp.bfloat16)
                s = s + jnp.dot(sl, wt, preferred_element_type=jnp.float32)
            if dx == -1:
                s = s * mask_l
            elif dx == 1:
                s = s * mask_r
            acc = acc + s
        return acc + bias

    p0_ref[pl.ds(0, PAD), :] = jnp.zeros((PAD, C), jnp.float32)
    p0_ref[pl.ds(P + PAD, PAD), :] = jnp.zeros((PAD, C), jnp.float32)
    p1_ref[...] = jnp.zeros_like(p1_ref)

    for bi in range(_EB):
        p1_ref[pl.ds(PAD, P), :] = x_ref[bi]
        x = jnp.maximum(conv1(p1_ref, w1_ref, b1_ref[...]), 0.0)

        p0_ref[pl.ds(PAD, P), :] = x
        y = jnp.maximum(conv_big(w2_ref, b2_ref[...]), 0.0)

        p0_ref[pl.ds(PAD, P), :] = y
        y = conv_big(w3_ref, b3_ref[...])

        out_ref[bi] = x + y


def _encoder(lr_rows, w1, b1, w2, b2, w3, b3):
    full = lambda s: pl.BlockSpec(s, lambda b: (0,) * len(s))
    return pl.pallas_call(
        _enc_body,
        grid=(B // _EB,),
        in_specs=[
            pl.BlockSpec((_EB, P, 3), lambda b: (b, 0, 0)),
            full((9, 3, C)), full((1, C)),
            full((9, C, C)), full((1, C)),
            full((9, C, C)), full((1, C)),
        ],
        out_specs=pl.BlockSpec((_EB, P, C), lambda b: (b, 0, 0)),
        out_shape=jax.ShapeDtypeStruct((B, P, C), jnp.float32),
        scratch_shapes=[
            pltpu.VMEM((P + 2 * PAD, 3), jnp.float32),
            pltpu.VMEM((P + 2 * PAD, C), jnp.float32),
        ],
        compiler_params=pltpu.CompilerParams(
            dimension_semantics=("parallel",)),
    )(lr_rows, w1, b1, w2, b2, w3, b3)


_NC = 2
_NS = 16
_NW = _NC * _NS
_RPW = R // _NW
_CHUNK = 128
_NCH = _RPW // _CHUNK


_SC_PARAMS = pltpu.CompilerParams(use_tc_tiling_on_sc=False)


def _sc_gather(feat_flat, lr_pad, idx_feat, idx_taps):

    @functools.partial(
        pl.kernel,
        mesh=plsc.VectorSubcoreMesh(core_axis_name="c", subcore_axis_name="s"),
        out_type=[
            jax.ShapeDtypeStruct((R, C), jnp.float32),
            jax.ShapeDtypeStruct((4, R, 8), jnp.float32),
        ],
        scratch_types=[
            pltpu.VMEM((_NCH, _CHUNK), jnp.int32),
            pltpu.VMEM((4, _NCH, _CHUNK), jnp.int32),
            pltpu.VMEM((_RPW, C), jnp.float32),
            pltpu.VMEM((4, _RPW, 8), jnp.float32),
            pltpu.SemaphoreType.DMA,
        ],
        compiler_params=_SC_PARAMS,
    )
    def k(feat_hbm, lr_hbm, idxf_hbm, idxt_hbm, qfeat_hbm, taps_hbm,
          idxf_v, idxt_v, rows_v, tap_v, sem):
        wid = lax.axis_index("s") * _NC + lax.axis_index("c")
        base = wid * _RPW
        pltpu.sync_copy(idxf_hbm.at[wid], idxf_v)
        for t in range(4):
            pltpu.sync_copy(idxt_hbm.at[t, wid], idxt_v.at[t])
        cps = []
        for j in range(_NCH):
            cps.append(pltpu.async_copy(
                feat_hbm.at[idxf_v.at[j]],
                rows_v.at[pl.ds(j * _CHUNK, _CHUNK)], sem))
        for t in range(4):
            for j in range(_NCH):
                cps.append(pltpu.async_copy(
                    lr_hbm.at[idxt_v.at[t, j]],
                    tap_v.at[t, pl.ds(j * _CHUNK, _CHUNK)], sem))
        for cp in cps:
            cp.wait()
        pltpu.sync_copy(rows_v, qfeat_hbm.at[pl.ds(base, _RPW)])
        for t in range(4):
            pltpu.sync_copy(tap_v.at[t], taps_hbm.at[t, pl.ds(base, _RPW)])

    return k(feat_flat, lr_pad, idx_feat, idx_taps)


_TILE = 2048


def _mlp_body(qf_ref, ex_ref, taps_ref, wts_ref,
              c1, cb1, c2, cb2, c3, cb3,
              l1, lb1, l2, lb2, l3, lb3,
              h1, hb1, h2, hb2, h3, hb3, h4, hb4, h5, hb5,
              pred_ref, prob_ref):
    qf = qf_ref[...]
    ex = ex_ref[...]
    qfh = qf.astype(jnp.bfloat16)
    exh = ex.astype(jnp.bfloat16)

    def dotf(a, b):
        return jnp.dot(a, b, preferred_element_type=jnp.float32)

    def dot(a, b):
        return jnp.dot(a.astype(jnp.bfloat16), b,
                       preferred_element_type=jnp.float32)

    def doth(a, b, bias):
        h = jnp.dot(a, b, preferred_element_type=jnp.float32)
        return jnp.maximum(h.astype(jnp.bfloat16) + bias.astype(jnp.bfloat16), 0)

    def first(w_ref, bias, lowp):
        if lowp:
            h = (jnp.dot(qfh, w_ref[:C], preferred_element_type=jnp.float32)
                 + jnp.dot(exh, w_ref[C:], preferred_element_type=jnp.float32))
            return jnp.maximum(h.astype(jnp.bfloat16) + bias.astype(jnp.bfloat16), 0)
        h = dotf(qf, w_ref[:C]) + dotf(ex, w_ref[C:])
        return jnp.maximum(h + bias, 0.0)

    hc = first(c1, cb1[...], False)
    hc = jnp.maximum(dotf(hc, c2[...]) + cb2[...], 0.0)
    logits = dotf(hc, c3[...]) + cb3[...]

    hl = first(l1, lb1[...], True)
    hl = doth(hl, l2[...], lb2[...])
    light = dot(hl, l3[...]) + lb3[...]

    hh = first(h1, hb1[...], True)
    hh = doth(hh, h2[...], hb2[...])
    hh = doth(hh, h3[...], hb3[...])
    hh = doth(hh, h4[...], hb4[...])
    heavy = dot(hh, h5[...]) + hb5[...]

    d = logits[:, 1:2] - logits[:, 0:1]
    p1 = 1.0 / (1.0 + jnp.exp(-d))
    p0 = 1.0 - p1
    prob_ref[...] = jnp.concatenate([p0, p1], axis=1)

    w = wts_ref[...]
    res = (w[:, 0:1] * taps_ref[0, :, 0:3] + w[:, 1:2] * taps_ref[1, :, 0:3]
           + w[:, 2:3] * taps_ref[2, :, 0:3] + w[:, 3:4] * taps_ref[3, :, 0:3])
    pred_ref[...] = p0 * light + p1 * heavy + res


def _mlp(q_feat, extra, taps, wts, cls_p, light_p, heavy_p):
    n = R // _TILE
    full = lambda s: pl.BlockSpec(s, lambda i: (0,) * len(s))
    wspecs = []
    wvals = []
    for p_list, lowp in ((cls_p, False), (light_p, True), (heavy_p, True)):
        for j in range(0, len(p_list), 2):
            wv, bv = p_list[j], p_list[j + 1]
            wvals += [wv.astype(jnp.bfloat16) if lowp else wv,
                      bv.reshape(1, -1)]
            wspecs += [full(wv.shape), full((1, bv.shape[0]))]
    return pl.pallas_call(
        _mlp_body,
        grid=(n,),
        in_specs=[
            pl.BlockSpec((_TILE, C), lambda i: (i, 0)),
            pl.BlockSpec((_TILE, 4), lambda i: (i, 0)),
            pl.BlockSpec((4, _TILE, 8), lambda i: (0, i, 0)),
            pl.BlockSpec((_TILE, 4), lambda i: (i, 0)),
        ] + wspecs,
        out_specs=[
            pl.BlockSpec((_TILE, 3), lambda i: (i, 0)),
            pl.BlockSpec((_TILE, 2), lambda i: (i, 0)),
        ],
        out_shape=[
            jax.ShapeDtypeStruct((R, 3), jnp.float32),
            jax.ShapeDtypeStruct((R, 2), jnp.float32),
        ],
        compiler_params=pltpu.CompilerParams(
            dimension_semantics=("parallel",)),
    )(q_feat, extra, taps, wts, *wvals)


def kernel(lr, coord, cell, enc_params, cls_params, light_params, heavy_params):
    lr_rows = jnp.transpose(lr, (0, 2, 3, 1)).reshape(B, P, 3)
    lr_pad = jnp.pad(lr_rows.reshape(R, 3), ((0, 0), (0, 5)))

    cy, cx = coord[..., 0], coord[..., 1]
    y = ((cy + 1.0) * H - 1.0) / 2.0
    x = ((cx + 1.0) * W - 1.0) / 2.0
    yi = jnp.clip(jnp.round(y), 0, H - 1).astype(jnp.int32)
    xi = jnp.clip(jnp.round(x), 0, W - 1).astype(jnp.int32)
    bbase = (jnp.arange(B, dtype=jnp.int32) * P)[:, None]
    idx_feat = (bbase + yi * W + xi).reshape(_NW, _NCH, _CHUNK)

    qy = ((yi.astype(jnp.float32) + 0.5) / H) * 2.0 - 1.0
    qx = ((xi.astype(jnp.float32) + 0.5) / W) * 2.0 - 1.0
    rel = jnp.stack([(cy - qy) * H, (cx - qx) * W], -1)
    relc = cell * jnp.array([H, W], jnp.float32)
    extra = jnp.concatenate([rel, relc], -1).reshape(R, 4)

    y0 = jnp.floor(y)
    x0 = jnp.floor(x)
    wy1 = y - y0
    wx1 = x - x0
    y0c = jnp.clip(y0, 0, H - 1).astype(jnp.int32)
    y1c = jnp.clip(y0 + 1.0, 0, H - 1).astype(jnp.int32)
    x0c = jnp.clip(x0, 0, W - 1).astype(jnp.int32)
    x1c = jnp.clip(x0 + 1.0, 0, W - 1).astype(jnp.int32)
    idx_taps = jnp.stack([
        bbase + y0c * W + x0c, bbase + y0c * W + x1c,
        bbase + y1c * W + x0c, bbase + y1c * W + x1c],
        0).reshape(4, _NW, _NCH, _CHUNK)
    wts = jnp.stack([(1 - wy1) * (1 - wx1), (1 - wy1) * wx1,
                     wy1 * (1 - wx1), wy1 * wx1], -1).reshape(R, 4)

    w1, b1, w2, b2, w3, b3 = enc_params
    w1r = jnp.transpose(w1, (2, 3, 1, 0)).reshape(9, 3, C)
    w2r = jnp.transpose(w2, (2, 3, 1, 0)).reshape(9, C, C)
    w3r = jnp.transpose(w3, (2, 3, 1, 0)).reshape(9, C, C)
    feat = _encoder(lr_rows, w1r, b1.reshape(1, C), w2r, b2.reshape(1, C),
                    w3r, b3.reshape(1, C))
    feat_flat = feat.reshape(R, C)

    q_feat, taps = _sc_gather(feat_flat, lr_pad, idx_feat, idx_taps)

    pred_flat, prob_flat = _mlp(q_feat, extra, taps, wts,
                                cls_params, light_params, heavy_params)
    return pred_flat.reshape(B, P, 3), prob_flat.reshape(B, P, 2)


# speedup vs baseline: 1.0757x; 1.0757x over previous
import functools

import jax
import jax.numpy as jnp
from jax import lax
from jax.experimental import pallas as pl
from jax.experimental.pallas import tpu as pltpu
from jax.experimental.pallas import tpu_sc as plsc

H = W = 48
P = H * W
B = 16
R = B * P
C = 64
PAD = 64
_EB = 2


def _enc_body(x_ref, w1_ref, b1_ref, w2_ref, b2_ref, w3_ref, b3_ref,
              out_ref, lrp_ref, p1_ref, p0_ref):
    col = lax.broadcasted_iota(jnp.int32, (P, 1), 0) % W
    mask_l = (col >= 1).astype(jnp.float32)
    mask_r = (col <= W - 2).astype(jnp.float32)

    def conv1(p_ref, w_ref, bias):
        acc = jnp.zeros((P, C), jnp.float32)
        for dx in (-1, 0, 1):
            s = jnp.zeros((P, C), jnp.float32)
            for dy in (-1, 0, 1):
                t = (dy + 1) * 3 + (dx + 1)
                sl = p_ref[pl.ds(PAD + dy * W + dx, P), :]
                s = s + jnp.dot(sl, w_ref[t], preferred_element_type=jnp.float32)
            if dx == -1:
                s = s * mask_l
            elif dx == 1:
                s = s * mask_r
            acc = acc + s
        return acc + bias

    def conv_big(w_ref, bias):
        acc = jnp.zeros((P, C), jnp.float32)
        for dx in (-1, 0, 1):
            s = jnp.zeros((P, C), jnp.float32)
            for dy in (-1, 0, 1):
                t = (dy + 1) * 3 + (dx + 1)
                sl = p0_ref[pl.ds(PAD + dy * W + dx, P), :].astype(jnp.bfloat16)
                wt = w_ref[t].astype(jnp.bfloat16)
                s = s + jnp.dot(sl, wt, preferred_element_type=jnp.float32)
            if dx == -1:
                s = s * mask_l
            elif dx == 1:
                s = s * mask_r
            acc = acc + s
        return acc + bias

    p0_ref[pl.ds(0, PAD), :] = jnp.zeros((PAD, C), jnp.float32)
    p0_ref[pl.ds(P + PAD, PAD), :] = jnp.zeros((PAD, C), jnp.float32)
    p1_ref[...] = jnp.zeros_like(p1_ref)

    for bi in range(_EB):
        rows = jnp.transpose(x_ref[bi].reshape(3, P), (1, 0))
        lrp_ref[bi] = jnp.pad(rows, ((0, 0), (0, 5)))
        p1_ref[pl.ds(PAD, P), :] = rows
        x = jnp.maximum(conv1(p1_ref, w1_ref, b1_ref[...]), 0.0)

        p0_ref[pl.ds(PAD, P), :] = x
        y = jnp.maximum(conv_big(w2_ref, b2_ref[...]), 0.0)

        p0_ref[pl.ds(PAD, P), :] = y
        y = conv_big(w3_ref, b3_ref[...])

        out_ref[bi] = x + y


def _encoder(lr, w1, b1, w2, b2, w3, b3):
    full = lambda s: pl.BlockSpec(s, lambda b: (0,) * len(s))
    return pl.pallas_call(
        _enc_body,
        grid=(B // _EB,),
        in_specs=[
            pl.BlockSpec((_EB, 3, H, W), lambda b: (b, 0, 0, 0)),
            full((9, 3, C)), full((1, C)),
            full((9, C, C)), full((1, C)),
            full((9, C, C)), full((1, C)),
        ],
        out_specs=[
            pl.BlockSpec((_EB, P, C), lambda b: (b, 0, 0)),
            pl.BlockSpec((_EB, P, 8), lambda b: (b, 0, 0)),
        ],
        out_shape=[
            jax.ShapeDtypeStruct((B, P, C), jnp.float32),
            jax.ShapeDtypeStruct((B, P, 8), jnp.float32),
        ],
        scratch_shapes=[
            pltpu.VMEM((P + 2 * PAD, 3), jnp.float32),
            pltpu.VMEM((P + 2 * PAD, C), jnp.float32),
        ],
        compiler_params=pltpu.CompilerParams(
            dimension_semantics=("parallel",)),
    )(lr, w1, b1, w2, b2, w3, b3)


_NC = 2
_NS = 16
_NW = _NC * _NS
_RPW = R // _NW
_CHUNK = 128
_NCH = _RPW // _CHUNK


_SC_PARAMS = pltpu.CompilerParams(use_tc_tiling_on_sc=False)


def _sc_gather(feat_flat, lr_pad, idx_feat, idx_taps):

    @functools.partial(
        pl.kernel,
        mesh=plsc.VectorSubcoreMesh(core_axis_name="c", subcore_axis_name="s"),
        out_type=[
            jax.ShapeDtypeStruct((R, C), jnp.float32),
            jax.ShapeDtypeStruct((4, R, 8), jnp.float32),
        ],
        scratch_types=[
            pltpu.VMEM((_NCH, _CHUNK), jnp.int32),
            pltpu.VMEM((4, _NCH, _CHUNK), jnp.int32),
            pltpu.VMEM((_RPW, C), jnp.float32),
            pltpu.VMEM((4, _RPW, 8), jnp.float32),
            pltpu.SemaphoreType.DMA,
        ],
        compiler_params=_SC_PARAMS,
    )
    def k(feat_hbm, lr_hbm, idxf_hbm, idxt_hbm, qfeat_hbm, taps_hbm,
          idxf_v, idxt_v, rows_v, tap_v, sem):
        wid = lax.axis_index("s") * _NC + lax.axis_index("c")
        base = wid * _RPW
        pltpu.sync_copy(idxf_hbm.at[wid], idxf_v)
        for t in range(4):
            pltpu.sync_copy(idxt_hbm.at[t, wid], idxt_v.at[t])
        cps = []
        for j in range(_NCH):
            cps.append(pltpu.async_copy(
                feat_hbm.at[idxf_v.at[j]],
                rows_v.at[pl.ds(j * _CHUNK, _CHUNK)], sem))
        for t in range(4):
            for j in range(_NCH):
                cps.append(pltpu.async_copy(
                    lr_hbm.at[idxt_v.at[t, j]],
                    tap_v.at[t, pl.ds(j * _CHUNK, _CHUNK)], sem))
        for cp in cps:
            cp.wait()
        pltpu.sync_copy(rows_v, qfeat_hbm.at[pl.ds(base, _RPW)])
        for t in range(4):
            pltpu.sync_copy(tap_v.at[t], taps_hbm.at[t, pl.ds(base, _RPW)])

    return k(feat_flat, lr_pad, idx_feat, idx_taps)


_TILE = 2048


def _mlp_body(qf_ref, ex_ref, taps_ref, wts_ref,
              c1, cb1, c2, cb2, c3, cb3,
              l1, lb1, l2, lb2, l3, lb3,
              h1, hb1, h2, hb2, h3, hb3, h4, hb4, h5, hb5,
              pred_ref, prob_ref):
    qf = qf_ref[...]
    ex = ex_ref[...]
    qfh = qf.astype(jnp.bfloat16)
    exh = ex.astype(jnp.bfloat16)

    def dotf(a, b):
        return jnp.dot(a, b, preferred_element_type=jnp.float32)

    def dot(a, b):
        return jnp.dot(a.astype(jnp.bfloat16), b,
                       preferred_element_type=jnp.float32)

    def doth(a, b, bias):
        h = jnp.dot(a, b, preferred_element_type=jnp.float32)
        return jnp.maximum(h.astype(jnp.bfloat16) + bias.astype(jnp.bfloat16), 0)

    def first(w_ref, bias, lowp):
        if lowp:
            h = (jnp.dot(qfh, w_ref[:C], preferred_element_type=jnp.float32)
                 + jnp.dot(exh, w_ref[C:], preferred_element_type=jnp.float32))
            return jnp.maximum(h.astype(jnp.bfloat16) + bias.astype(jnp.bfloat16), 0)
        h = dotf(qf, w_ref[:C]) + dotf(ex, w_ref[C:])
        return jnp.maximum(h + bias, 0.0)

    hc = first(c1, cb1[...], False)
    hc = jnp.maximum(dotf(hc, c2[...]) + cb2[...], 0.0)
    logits = dotf(hc, c3[...]) + cb3[...]

    hl = first(l1, lb1[...], True)
    hl = doth(hl, l2[...], lb2[...])
    light = dot(hl, l3[...]) + lb3[...]

    hh = first(h1, hb1[...], True)
    hh = doth(hh, h2[...], hb2[...])
    hh = doth(hh, h3[...], hb3[...])
    hh = doth(hh, h4[...], hb4[...])
    heavy = dot(hh, h5[...]) + hb5[...]

    d = logits[:, 1:2] - logits[:, 0:1]
    p1 = 1.0 / (1.0 + jnp.exp(-d))
    p0 = 1.0 - p1
    prob_ref[...] = jnp.concatenate([p0, p1], axis=1)

    w = wts_ref[...]
    res = (w[:, 0:1] * taps_ref[0, :, 0:3] + w[:, 1:2] * taps_ref[1, :, 0:3]
           + w[:, 2:3] * taps_ref[2, :, 0:3] + w[:, 3:4] * taps_ref[3, :, 0:3])
    pred_ref[...] = p0 * light + p1 * heavy + res


def _mlp(q_feat, extra, taps, wts, cls_p, light_p, heavy_p):
    n = R // _TILE
    full = lambda s: pl.BlockSpec(s, lambda i: (0,) * len(s))
    wspecs = []
    wvals = []
    for p_list, lowp in ((cls_p, False), (light_p, True), (heavy_p, True)):
        for j in range(0, len(p_list), 2):
            wv, bv = p_list[j], p_list[j + 1]
            wvals += [wv.astype(jnp.bfloat16) if lowp else wv,
                      bv.reshape(1, -1)]
            wspecs += [full(wv.shape), full((1, bv.shape[0]))]
    return pl.pallas_call(
        _mlp_body,
        grid=(n,),
        in_specs=[
            pl.BlockSpec((_TILE, C), lambda i: (i, 0)),
            pl.BlockSpec((_TILE, 4), lambda i: (i, 0)),
            pl.BlockSpec((4, _TILE, 8), lambda i: (0, i, 0)),
            pl.BlockSpec((_TILE, 4), lambda i: (i, 0)),
        ] + wspecs,
        out_specs=[
            pl.BlockSpec((_TILE, 3), lambda i: (i, 0)),
            pl.BlockSpec((_TILE, 2), lambda i: (i, 0)),
        ],
        out_shape=[
            jax.ShapeDtypeStruct((R, 3), jnp.float32),
            jax.ShapeDtypeStruct((R, 2), jnp.float32),
        ],
        compiler_params=pltpu.CompilerParams(
            dimension_semantics=("parallel",)),
    )(q_feat, extra, taps, wts, *wvals)


def kernel(lr, coord, cell, enc_params, cls_params, light_params, heavy_params):
    cy, cx = coord[..., 0], coord[..., 1]
    y = ((cy + 1.0) * H - 1.0) / 2.0
    x = ((cx + 1.0) * W - 1.0) / 2.0
    yi = jnp.clip(jnp.round(y), 0, H - 1).astype(jnp.int32)
    xi = jnp.clip(jnp.round(x), 0, W - 1).astype(jnp.int32)
    bbase = (jnp.arange(B, dtype=jnp.int32) * P)[:, None]
    idx_feat = (bbase + yi * W + xi).reshape(_NW, _NCH, _CHUNK)

    qy = ((yi.astype(jnp.float32) + 0.5) / H) * 2.0 - 1.0
    qx = ((xi.astype(jnp.float32) + 0.5) / W) * 2.0 - 1.0
    rel = jnp.stack([(cy - qy) * H, (cx - qx) * W], -1)
    relc = cell * jnp.array([H, W], jnp.float32)
    extra = jnp.concatenate([rel, relc], -1).reshape(R, 4)

    y0 = jnp.floor(y)
    x0 = jnp.floor(x)
    wy1 = y - y0
    wx1 = x - x0
    y0c = jnp.clip(y0, 0, H - 1).astype(jnp.int32)
    y1c = jnp.clip(y0 + 1.0, 0, H - 1).astype(jnp.int32)
    x0c = jnp.clip(x0, 0, W - 1).astype(jnp.int32)
    x1c = jnp.clip(x0 + 1.0, 0, W - 1).astype(jnp.int32)
    idx_taps = jnp.stack([
        bbase + y0c * W + x0c, bbase + y0c * W + x1c,
        bbase + y1c * W + x0c, bbase + y1c * W + x1c],
        0).reshape(4, _NW, _NCH, _CHUNK)
    wts = jnp.stack([(1 - wy1) * (1 - wx1), (1 - wy1) * wx1,
                     wy1 * (1 - wx1), wy1 * wx1], -1).reshape(R, 4)

    w1, b1, w2, b2, w3, b3 = enc_params
    w1r = jnp.transpose(w1, (2, 3, 1, 0)).reshape(9, 3, C)
    w2r = jnp.transpose(w2, (2, 3, 1, 0)).reshape(9, C, C)
    w3r = jnp.transpose(w3, (2, 3, 1, 0)).reshape(9, C, C)
    feat, lr_pad8 = _encoder(lr, w1r, b1.reshape(1, C), w2r, b2.reshape(1, C),
                             w3r, b3.reshape(1, C))
    feat_flat = feat.reshape(R, C)
    lr_pad = lr_pad8.reshape(R, 8)

    q_feat, taps = _sc_gather(feat_flat, lr_pad, idx_feat, idx_taps)

    pred_flat, prob_flat = _mlp(q_feat, extra, taps, wts,
                                cls_params, light_params, heavy_params)
    return pred_flat.reshape(B, P, 3), prob_flat.reshape(B, P, 2)


# speedup vs baseline: 1.1067x; 1.0288x over previous
import functools

import jax
import jax.numpy as jnp
from jax import lax
from jax.experimental import pallas as pl
from jax.experimental.pallas import tpu as pltpu
from jax.experimental.pallas import tpu_sc as plsc

H = W = 48
P = H * W
B = 16
R = B * P
C = 64
PAD = 64
_EB = 2


def _enc_body(x_ref, w1_ref, b1_ref, w2_ref, b2_ref, w3_ref, b3_ref,
              out_ref, lrp_ref, p1_ref, p0_ref):
    col = lax.broadcasted_iota(jnp.int32, (P, 1), 0) % W
    mask_l = (col >= 1).astype(jnp.float32)
    mask_r = (col <= W - 2).astype(jnp.float32)

    def conv1(p_ref, w_ref, bias):
        acc = jnp.zeros((P, C), jnp.float32)
        for dx in (-1, 0, 1):
            s = jnp.zeros((P, C), jnp.float32)
            for dy in (-1, 0, 1):
                t = (dy + 1) * 3 + (dx + 1)
                sl = p_ref[pl.ds(PAD + dy * W + dx, P), :]
                s = s + jnp.dot(sl, w_ref[t], preferred_element_type=jnp.float32)
            if dx == -1:
                s = s * mask_l
            elif dx == 1:
                s = s * mask_r
            acc = acc + s
        return acc + bias

    def conv_big(w_ref, bias):
        acc = jnp.zeros((P, C), jnp.float32)
        for dx in (-1, 0, 1):
            s = jnp.zeros((P, C), jnp.float32)
            for dy in (-1, 0, 1):
                t = (dy + 1) * 3 + (dx + 1)
                sl = p0_ref[pl.ds(PAD + dy * W + dx, P), :].astype(jnp.bfloat16)
                wt = w_ref[t].astype(jnp.bfloat16)
                s = s + jnp.dot(sl, wt, preferred_element_type=jnp.float32)
            if dx == -1:
                s = s * mask_l
            elif dx == 1:
                s = s * mask_r
            acc = acc + s
        return acc + bias

    p0_ref[pl.ds(0, PAD), :] = jnp.zeros((PAD, C), jnp.float32)
    p0_ref[pl.ds(P + PAD, PAD), :] = jnp.zeros((PAD, C), jnp.float32)
    p1_ref[...] = jnp.zeros_like(p1_ref)

    for bi in range(_EB):
        rows = jnp.transpose(x_ref[bi].reshape(3, P), (1, 0))
        lrp_ref[bi] = jnp.pad(rows, ((0, 0), (0, 5)))
        p1_ref[pl.ds(PAD, P), :] = rows
        x = jnp.maximum(conv1(p1_ref, w1_ref, b1_ref[...]), 0.0)

        p0_ref[pl.ds(PAD, P), :] = x
        y = jnp.maximum(conv_big(w2_ref, b2_ref[...]), 0.0)

        p0_ref[pl.ds(PAD, P), :] = y
        y = conv_big(w3_ref, b3_ref[...])

        out_ref[bi] = x + y


def _encoder(lr, w1, b1, w2, b2, w3, b3):
    full = lambda s: pl.BlockSpec(s, lambda b: (0,) * len(s))
    return pl.pallas_call(
        _enc_body,
        grid=(B // _EB,),
        in_specs=[
            pl.BlockSpec((_EB, 3, H, W), lambda b: (b, 0, 0, 0)),
            full((9, 3, C)), full((1, C)),
            full((9, C, C)), full((1, C)),
            full((9, C, C)), full((1, C)),
        ],
        out_specs=[
            pl.BlockSpec((_EB, P, C), lambda b: (b, 0, 0)),
            pl.BlockSpec((_EB, P, 8), lambda b: (b, 0, 0)),
        ],
        out_shape=[
            jax.ShapeDtypeStruct((B, P, C), jnp.float32),
            jax.ShapeDtypeStruct((B, P, 8), jnp.float32),
        ],
        scratch_shapes=[
            pltpu.VMEM((P + 2 * PAD, 3), jnp.float32),
            pltpu.VMEM((P + 2 * PAD, C), jnp.float32),
        ],
        compiler_params=pltpu.CompilerParams(
            dimension_semantics=("parallel",)),
    )(lr, w1, b1, w2, b2, w3, b3)


_NC = 2
_NS = 16
_NW = _NC * _NS
_RPW = R // _NW
_CHUNK = 128
_NCH = _RPW // _CHUNK


_SC_PARAMS = pltpu.CompilerParams(use_tc_tiling_on_sc=False)


def _sc_gather(feat_flat, lr_pad, idx_all):

    @functools.partial(
        pl.kernel,
        mesh=plsc.VectorSubcoreMesh(core_axis_name="c", subcore_axis_name="s"),
        out_type=[
            jax.ShapeDtypeStruct((R, C), jnp.float32),
            jax.ShapeDtypeStruct((4, R, 8), jnp.float32),
        ],
        scratch_types=[
            pltpu.VMEM((5, _NCH, _CHUNK), jnp.int32),
            pltpu.VMEM((_RPW, C), jnp.float32),
            pltpu.VMEM((4, _RPW, 8), jnp.float32),
            pltpu.SemaphoreType.DMA,
        ],
        compiler_params=_SC_PARAMS,
    )
    def k(feat_hbm, lr_hbm, idxa_hbm, qfeat_hbm, taps_hbm,
          idxa_v, rows_v, tap_v, sem):
        wid = lax.axis_index("s") * _NC + lax.axis_index("c")
        base = wid * _RPW
        for t in range(5):
            pltpu.sync_copy(idxa_hbm.at[t, wid], idxa_v.at[t])
        cps = []
        for j in range(_NCH):
            cps.append(pltpu.async_copy(
                feat_hbm.at[idxa_v.at[0, j]],
                rows_v.at[pl.ds(j * _CHUNK, _CHUNK)], sem))
        for t in range(4):
            for j in range(_NCH):
                cps.append(pltpu.async_copy(
                    lr_hbm.at[idxa_v.at[1 + t, j]],
                    tap_v.at[t, pl.ds(j * _CHUNK, _CHUNK)], sem))
        for cp in cps:
            cp.wait()
        pltpu.sync_copy(rows_v, qfeat_hbm.at[pl.ds(base, _RPW)])
        for t in range(4):
            pltpu.sync_copy(tap_v.at[t], taps_hbm.at[t, pl.ds(base, _RPW)])

    return k(feat_flat, lr_pad, idx_all)


_TILE = 2048


def _mlp_body(qf_ref, ew_ref, taps_ref,
              c1, cb1, c2, cb2, c3, cb3,
              l1, lb1, l2, lb2, l3, lb3,
              h1, hb1, h2, hb2, h3, hb3, h4, hb4, h5, hb5,
              pred_ref, prob_ref):
    qf = qf_ref[...]
    ew = ew_ref[...]
    ex = ew[:, 0:4]
    qfh = qf.astype(jnp.bfloat16)
    exh = ex.astype(jnp.bfloat16)

    def dotf(a, b):
        return jnp.dot(a, b, preferred_element_type=jnp.float32)

    def dot(a, b):
        return jnp.dot(a.astype(jnp.bfloat16), b,
                       preferred_element_type=jnp.float32)

    def doth(a, b, bias):
        h = jnp.dot(a, b, preferred_element_type=jnp.float32)
        return jnp.maximum(h.astype(jnp.bfloat16) + bias.astype(jnp.bfloat16), 0)

    def first(w_ref, bias, lowp):
        if lowp:
            h = (jnp.dot(qfh, w_ref[:C], preferred_element_type=jnp.float32)
                 + jnp.dot(exh, w_ref[C:], preferred_element_type=jnp.float32))
            return jnp.maximum(h.astype(jnp.bfloat16) + bias.astype(jnp.bfloat16), 0)
        h = dotf(qf, w_ref[:C]) + dotf(ex, w_ref[C:])
        return jnp.maximum(h + bias, 0.0)

    hc = first(c1, cb1[...], False)
    hc = jnp.maximum(dotf(hc, c2[...]) + cb2[...], 0.0)
    logits = dotf(hc, c3[...]) + cb3[...]

    hl = first(l1, lb1[...], True)
    hl = doth(hl, l2[...], lb2[...])
    light = dot(hl, l3[...]) + lb3[...]

    hh = first(h1, hb1[...], True)
    hh = doth(hh, h2[...], hb2[...])
    hh = doth(hh, h3[...], hb3[...])
    hh = doth(hh, h4[...], hb4[...])
    heavy = dot(hh, h5[...]) + hb5[...]

    d = logits[:, 1:2] - logits[:, 0:1]
    p1 = 1.0 / (1.0 + jnp.exp(-d))
    p0 = 1.0 - p1
    prob_ref[...] = jnp.concatenate([p0, p1], axis=1)

    res = (ew[:, 4:5] * taps_ref[0, :, 0:3] + ew[:, 5:6] * taps_ref[1, :, 0:3]
           + ew[:, 6:7] * taps_ref[2, :, 0:3] + ew[:, 7:8] * taps_ref[3, :, 0:3])
    pred_ref[...] = p0 * light + p1 * heavy + res


def _mlp(q_feat, ew, taps, cls_p, light_p, heavy_p):
    n = R // _TILE
    full = lambda s: pl.BlockSpec(s, lambda i: (0,) * len(s))
    wspecs = []
    wvals = []
    for p_list, lowp in ((cls_p, False), (light_p, True), (heavy_p, True)):
        for j in range(0, len(p_list), 2):
            wv, bv = p_list[j], p_list[j + 1]
            wvals += [wv.astype(jnp.bfloat16) if lowp else wv,
                      bv.reshape(1, -1)]
            wspecs += [full(wv.shape), full((1, bv.shape[0]))]
    return pl.pallas_call(
        _mlp_body,
        grid=(n,),
        in_specs=[
            pl.BlockSpec((_TILE, C), lambda i: (i, 0)),
            pl.BlockSpec((_TILE, 8), lambda i: (i, 0)),
            pl.BlockSpec((4, _TILE, 8), lambda i: (0, i, 0)),
        ] + wspecs,
        out_specs=[
            pl.BlockSpec((_TILE, 3), lambda i: (i, 0)),
            pl.BlockSpec((_TILE, 2), lambda i: (i, 0)),
        ],
        out_shape=[
            jax.ShapeDtypeStruct((R, 3), jnp.float32),
            jax.ShapeDtypeStruct((R, 2), jnp.float32),
        ],
        compiler_params=pltpu.CompilerParams(
            dimension_semantics=("parallel",)),
    )(q_feat, ew, taps, *wvals)


def kernel(lr, coord, cell, enc_params, cls_params, light_params, heavy_params):
    cy, cx = coord[..., 0], coord[..., 1]
    y = ((cy + 1.0) * H - 1.0) / 2.0
    x = ((cx + 1.0) * W - 1.0) / 2.0
    yi = jnp.clip(jnp.round(y), 0, H - 1).astype(jnp.int32)
    xi = jnp.clip(jnp.round(x), 0, W - 1).astype(jnp.int32)
    bbase = (jnp.arange(B, dtype=jnp.int32) * P)[:, None]

    y0 = jnp.floor(y)
    x0 = jnp.floor(x)
    wy1 = y - y0
    wx1 = x - x0
    y0c = jnp.clip(y0, 0, H - 1).astype(jnp.int32)
    y1c = jnp.clip(y0 + 1.0, 0, H - 1).astype(jnp.int32)
    x0c = jnp.clip(x0, 0, W - 1).astype(jnp.int32)
    x1c = jnp.clip(x0 + 1.0, 0, W - 1).astype(jnp.int32)
    idx_all = jnp.stack([
        bbase + yi * W + xi,
        bbase + y0c * W + x0c, bbase + y0c * W + x1c,
        bbase + y1c * W + x0c, bbase + y1c * W + x1c],
        0).reshape(5, _NW, _NCH, _CHUNK)

    qy = ((yi.astype(jnp.float32) + 0.5) / H) * 2.0 - 1.0
    qx = ((xi.astype(jnp.float32) + 0.5) / W) * 2.0 - 1.0
    ew = jnp.stack([
        (cy - qy) * H, (cx - qx) * W,
        cell[..., 0] * H, cell[..., 1] * W,
        (1 - wy1) * (1 - wx1), (1 - wy1) * wx1,
        wy1 * (1 - wx1), wy1 * wx1], -1).reshape(R, 8)

    w1, b1, w2, b2, w3, b3 = enc_params
    w1r = jnp.transpose(w1, (2, 3, 1, 0)).reshape(9, 3, C)
    w2r = jnp.transpose(w2, (2, 3, 1, 0)).reshape(9, C, C)
    w3r = jnp.transpose(w3, (2, 3, 1, 0)).reshape(9, C, C)
    feat, lr_pad8 = _encoder(lr, w1r, b1.reshape(1, C), w2r, b2.reshape(1, C),
                             w3r, b3.reshape(1, C))
    feat_flat = feat.reshape(R, C)
    lr_pad = lr_pad8.reshape(R, 8)

    q_feat, taps = _sc_gather(feat_flat, lr_pad, idx_all)

    pred_flat, prob_flat = _mlp(q_feat, ew, taps,
                                cls_params, light_params, heavy_params)
    return pred_flat.reshape(B, P, 3), prob_flat.reshape(B, P, 2)
